# baseline (speedup 1.0000x reference)
import functools

import jax
import jax.numpy as jnp
from jax import lax
from jax.experimental import pallas as pl
from jax.experimental.pallas import tpu as pltpu

NUM_EMBEDDINGS = 1024
EMBEDDING_DIM = 256
COMMITMENT_COST = 0.25


def _vq_kernel(x_ref, w_ref, enc_ref, qz_ref, counts_ref, loss_ref,
               ppl_ref, wm2_scr, w2_scr, acc, *, n_rows_total, grid_r):
    r = pl.program_id(0)

    @pl.when(r == 0)
    def _():
        w = w_ref[:]
        wm2_scr[:] = w * jnp.float32(-2.0)
        w2_scr[0, :] = jnp.sum(w * w, axis=1)

    xb = x_ref[:]

    x2 = jnp.sum(xb * xb, axis=1)
    mm = lax.dot_general(xb, wm2_scr[:], (((1,), (1,)), ((), ())),
                         preferred_element_type=jnp.float32)
    d = (x2[:, None] + w2_scr[0, :][None, :]) + mm

    idx = jnp.argmin(d, axis=1)
    kiota = lax.broadcasted_iota(jnp.int32, (1, NUM_EMBEDDINGS), 1)
    onehot = (kiota == idx[:, None]).astype(jnp.float32)
    enc_ref[:] = onehot

    q = lax.dot_general(onehot, w_ref[:], (((1,), (0,)), ((), ())),
                        preferred_element_type=jnp.float32)
    qz_ref[:] = q

    part_x2 = jnp.sum(x2)
    part_xq = jnp.sum(xb * q)
    part_counts = jnp.sum(onehot, axis=0, keepdims=True)

    @pl.when(r == 0)
    def _():
        acc[0, 0] = part_x2
        acc[0, 1] = part_xq
        counts_ref[:] = part_counts

    @pl.when(r != 0)
    def _():
        acc[0, 0] = acc[0, 0] + part_x2
        acc[0, 1] = acc[0, 1] + part_xq
        counts_ref[:] = counts_ref[:] + part_counts

    @pl.when(r == grid_r - 1)
    def _():
        counts = counts_ref[:]
        sse = (acc[0, 0] + jnp.sum(counts * w2_scr[:])
               - 2.0 * acc[0, 1])
        mse = sse / (n_rows_total * EMBEDDING_DIM)
        loss_ref[0, 0] = (1.0 + COMMITMENT_COST) * mse
        probs = counts / n_rows_total
        ent = -jnp.sum(probs * jnp.log(probs + 1e-10))
        ppl_ref[0, 0] = jnp.exp(ent)


def kernel(x, weight, reset):
    B, C, H, W = x.shape
    n_rows_total = B * H * W
    R = 512
    grid_r = n_rows_total // R
    xf = jnp.transpose(x, (0, 2, 3, 1)).reshape(n_rows_total, C)

    body = functools.partial(_vq_kernel, n_rows_total=float(n_rows_total),
                             grid_r=grid_r)
    enc, qzf, counts, loss, ppl = pl.pallas_call(
        body,
        grid=(grid_r,),
        in_specs=[
            pl.BlockSpec((R, C), lambda r: (r, 0)),
            pl.BlockSpec((NUM_EMBEDDINGS, C), lambda r: (0, 0)),
        ],
        out_specs=[
            pl.BlockSpec((R, NUM_EMBEDDINGS), lambda r: (r, 0)),
            pl.BlockSpec((R, C), lambda r: (r, 0)),
            pl.BlockSpec((1, NUM_EMBEDDINGS), lambda r: (0, 0)),
            pl.BlockSpec(memory_space=pltpu.SMEM),
            pl.BlockSpec(memory_space=pltpu.SMEM),
        ],
        out_shape=[
            jax.ShapeDtypeStruct((n_rows_total, NUM_EMBEDDINGS), jnp.float32),
            jax.ShapeDtypeStruct((n_rows_total, C), jnp.float32),
            jax.ShapeDtypeStruct((1, NUM_EMBEDDINGS), jnp.float32),
            jax.ShapeDtypeStruct((1, 1), jnp.float32),
            jax.ShapeDtypeStruct((1, 1), jnp.float32),
        ],
        scratch_shapes=[pltpu.VMEM((NUM_EMBEDDINGS, C), jnp.float32),
                        pltpu.VMEM((1, NUM_EMBEDDINGS), jnp.float32),
                        pltpu.SMEM((1, 2), jnp.float32)],
    )(xf, weight)
    qz = jnp.transpose(qzf.reshape(B, H, W, C), (0, 3, 1, 2))
    return (loss[0, 0], qz, ppl[0, 0], enc)


# speedup vs baseline: 1.2795x; 1.2795x over previous
import functools

import jax
import jax.numpy as jnp
from jax import lax
from jax.experimental import pallas as pl
from jax.experimental.pallas import tpu as pltpu

NUM_EMBEDDINGS = 1024
EMBEDDING_DIM = 256
COMMITMENT_COST = 0.25


def _vq_kernel(x_ref, w_ref, enc_ref, qz_ref, counts_ref, loss_ref,
               ppl_ref, wm2_scr, w2_scr, loss_acc, *, n_rows_total, grid_r):
    r = pl.program_id(0)

    @pl.when(r == 0)
    def _():
        w = w_ref[:]
        wm2_scr[:] = w * jnp.float32(-2.0)
        w2_scr[0, :] = jnp.sum(w * w, axis=1)

    xb = x_ref[:]

    x2 = jnp.sum(xb * xb, axis=1)
    mm = lax.dot_general(xb, wm2_scr[:], (((1,), (1,)), ((), ())),
                         preferred_element_type=jnp.float32)
    d = (x2[:, None] + w2_scr[0, :][None, :]) + mm

    dmin = jnp.min(d, axis=1, keepdims=True)
    fiota = lax.broadcasted_iota(
        jnp.int32, (1, NUM_EMBEDDINGS), 1).astype(jnp.float32)
    masked = jnp.where(d == dmin, fiota, jnp.float32(2.0e9))
    idxf = jnp.min(masked, axis=1, keepdims=True)
    onehot = (masked == idxf).astype(jnp.float32)
    enc_ref[:] = onehot

    qz_ref[:] = lax.dot_general(onehot, w_ref[:], (((1,), (0,)), ((), ())),
                                preferred_element_type=jnp.float32)

    part_loss = jnp.sum(dmin)
    part_counts = jnp.sum(onehot, axis=0, keepdims=True)

    @pl.when(r == 0)
    def _():
        loss_acc[0, 0] = part_loss
        counts_ref[:] = part_counts

    @pl.when(r != 0)
    def _():
        loss_acc[0, 0] = loss_acc[0, 0] + part_loss
        counts_ref[:] = counts_ref[:] + part_counts

    @pl.when(r == grid_r - 1)
    def _():
        mse = loss_acc[0, 0] / (n_rows_total * EMBEDDING_DIM)
        loss_ref[0, 0] = (1.0 + COMMITMENT_COST) * mse
        probs = counts_ref[:] / n_rows_total
        ent = -jnp.sum(probs * jnp.log(probs + 1e-10))
        ppl_ref[0, 0] = jnp.exp(ent)


def kernel(x, weight, reset):
    B, C, H, W = x.shape
    n_rows_total = B * H * W
    R = 1024
    grid_r = n_rows_total // R
    xf = jnp.transpose(x, (0, 2, 3, 1)).reshape(n_rows_total, C)

    body = functools.partial(_vq_kernel, n_rows_total=float(n_rows_total),
                             grid_r=grid_r)
    enc, qzf, counts, loss, ppl = pl.pallas_call(
        body,
        grid=(grid_r,),
        in_specs=[
            pl.BlockSpec((R, C), lambda r: (r, 0)),
            pl.BlockSpec((NUM_EMBEDDINGS, C), lambda r: (0, 0)),
        ],
        out_specs=[
            pl.BlockSpec((R, NUM_EMBEDDINGS), lambda r: (r, 0)),
            pl.BlockSpec((R, C), lambda r: (r, 0)),
            pl.BlockSpec((1, NUM_EMBEDDINGS), lambda r: (0, 0)),
            pl.BlockSpec(memory_space=pltpu.SMEM),
            pl.BlockSpec(memory_space=pltpu.SMEM),
        ],
        out_shape=[
            jax.ShapeDtypeStruct((n_rows_total, NUM_EMBEDDINGS), jnp.float32),
            jax.ShapeDtypeStruct((n_rows_total, C), jnp.float32),
            jax.ShapeDtypeStruct((1, NUM_EMBEDDINGS), jnp.float32),
            jax.ShapeDtypeStruct((1, 1), jnp.float32),
            jax.ShapeDtypeStruct((1, 1), jnp.float32),
        ],
        scratch_shapes=[pltpu.VMEM((NUM_EMBEDDINGS, C), jnp.float32),
                        pltpu.VMEM((1, NUM_EMBEDDINGS), jnp.float32),
                        pltpu.SMEM((1, 1), jnp.float32)],
    )(xf, weight)
    qz = jnp.transpose(qzf.reshape(B, H, W, C), (0, 3, 1, 2))
    return (loss[0, 0], qz, ppl[0, 0], enc)


# speedup vs baseline: 1.3503x; 1.0554x over previous
import functools

import jax
import jax.numpy as jnp
from jax import lax
from jax.experimental import pallas as pl
from jax.experimental.pallas import tpu as pltpu

NUM_EMBEDDINGS = 1024
EMBEDDING_DIM = 256
COMMITMENT_COST = 0.25


def _vq_kernel(x_ref, w_ref, enc_ref, qz_ref, counts_ref, loss_ref,
               ppl_ref, wm2_scr, w2_scr, loss_acc, *, n_rows_total, grid_r):
    r = pl.program_id(0)

    @pl.when(r == 0)
    def _():
        w = w_ref[:]
        wm2_scr[:] = w * jnp.float32(-2.0)
        w2_scr[0, :] = jnp.sum(w * w, axis=1)

    xb = x_ref[:]

    x2 = jnp.sum(xb * xb, axis=1)
    mm = lax.dot_general(xb, wm2_scr[:], (((1,), (1,)), ((), ())),
                         preferred_element_type=jnp.float32)
    d = (x2[:, None] + w2_scr[0, :][None, :]) + mm

    dmin = jnp.min(d, axis=1, keepdims=True)
    fiota = lax.broadcasted_iota(
        jnp.int32, (1, NUM_EMBEDDINGS), 1).astype(jnp.float32)
    masked = jnp.where(d == dmin, fiota, jnp.float32(2.0e9))
    idxf = jnp.min(masked, axis=1, keepdims=True)
    onehot = (masked == idxf).astype(jnp.float32)
    enc_ref[:] = onehot

    qz_ref[:] = lax.dot_general(onehot, w_ref[:], (((1,), (0,)), ((), ())),
                                preferred_element_type=jnp.float32)

    part_loss = jnp.sum(dmin)
    part_counts = jnp.sum(onehot, axis=0, keepdims=True)

    @pl.when(r == 0)
    def _():
        loss_acc[0, 0] = part_loss
        counts_ref[:] = part_counts

    @pl.when(r != 0)
    def _():
        loss_acc[0, 0] = loss_acc[0, 0] + part_loss
        counts_ref[:] = counts_ref[:] + part_counts

    @pl.when(r == grid_r - 1)
    def _():
        mse = loss_acc[0, 0] / (n_rows_total * EMBEDDING_DIM)
        loss_ref[0, 0] = (1.0 + COMMITMENT_COST) * mse
        probs = counts_ref[:] / n_rows_total
        ent = -jnp.sum(probs * jnp.log(probs + 1e-10))
        ppl_ref[0, 0] = jnp.exp(ent)


def kernel(x, weight, reset):
    B, C, H, W = x.shape
    n_rows_total = B * H * W
    R = 2048
    grid_r = n_rows_total // R
    xf = jnp.transpose(x, (0, 2, 3, 1)).reshape(n_rows_total, C)

    body = functools.partial(_vq_kernel, n_rows_total=float(n_rows_total),
                             grid_r=grid_r)
    enc, qzf, counts, loss, ppl = pl.pallas_call(
        body,
        grid=(grid_r,),
        in_specs=[
            pl.BlockSpec((R, C), lambda r: (r, 0)),
            pl.BlockSpec((NUM_EMBEDDINGS, C), lambda r: (0, 0)),
        ],
        out_specs=[
            pl.BlockSpec((R, NUM_EMBEDDINGS), lambda r: (r, 0)),
            pl.BlockSpec((R, C), lambda r: (r, 0)),
            pl.BlockSpec((1, NUM_EMBEDDINGS), lambda r: (0, 0)),
            pl.BlockSpec(memory_space=pltpu.SMEM),
            pl.BlockSpec(memory_space=pltpu.SMEM),
        ],
        out_shape=[
            jax.ShapeDtypeStruct((n_rows_total, NUM_EMBEDDINGS), jnp.float32),
            jax.ShapeDtypeStruct((n_rows_total, C), jnp.float32),
            jax.ShapeDtypeStruct((1, NUM_EMBEDDINGS), jnp.float32),
            jax.ShapeDtypeStruct((1, 1), jnp.float32),
            jax.ShapeDtypeStruct((1, 1), jnp.float32),
        ],
        scratch_shapes=[pltpu.VMEM((NUM_EMBEDDINGS, C), jnp.float32),
                        pltpu.VMEM((1, NUM_EMBEDDINGS), jnp.float32),
                        pltpu.SMEM((1, 1), jnp.float32)],
    )(xf, weight)
    qz = jnp.transpose(qzf.reshape(B, H, W, C), (0, 3, 1, 2))
    return (loss[0, 0], qz, ppl[0, 0], enc)
